# baseline (speedup 1.0000x reference)
import functools

import jax
import jax.numpy as jnp
from jax import lax
from jax.experimental import pallas as pl
from jax.experimental.pallas import tpu as pltpu
from jax.experimental.pallas import tpu_sc as plsc

N_CTX = 8
CTX_LEN = 77
CTX_DIM = 768
BATCH = 1024
NUM_WORKERS = 32
B_PER_W = BATCH // NUM_WORKERS


N_SUF = CTX_LEN - 1 - N_CTX


SUF0 = 1 + N_CTX


def _prompt_body(tok_hbm, sos_hbm, suf_hbm, table_hbm, ctxp_hbm, ctxn_hbm,
                 out_hbm, tokout_hbm,
                 tokblk_v, sos_v, suf_v, buf0, buf1, ctxn_v,
                 sem_g0, sem_g1, sem_w0, sem_w1):
    wid = lax.axis_index("s") * 2 + lax.axis_index("c")
    base = wid * B_PER_W

    pltpu.sync_copy(ctxp_hbm.at[0], buf0.at[pl.ds(1, N_CTX)])
    pltpu.sync_copy(ctxp_hbm.at[0], buf1.at[pl.ds(1, N_CTX)])
    pltpu.sync_copy(ctxn_hbm.at[0], ctxn_v)
    pltpu.sync_copy(tok_hbm.at[pl.ds(base, B_PER_W)], tokblk_v)
    pltpu.sync_copy(sos_hbm.at[pl.ds(base, B_PER_W)], sos_v)
    pltpu.sync_copy(suf_hbm.at[pl.ds(base, B_PER_W)], suf_v)

    pltpu.sync_copy(tokblk_v, tokout_hbm.at[pl.ds(base, B_PER_W)])
    pltpu.sync_copy(tokblk_v, tokout_hbm.at[pl.ds(base + BATCH, B_PER_W)])

    def gather(li, buf, sem):
        return (
            pltpu.async_copy(table_hbm.at[sos_v.at[li]],
                             buf.at[pl.ds(0, 1)], sem),
            pltpu.async_copy(table_hbm.at[suf_v.at[li]],
                             buf.at[pl.ds(SUF0, N_SUF)], sem),
        )

    def write(b, buf, sem):
        return (
            pltpu.async_copy(buf, out_hbm.at[b], sem),
            pltpu.async_copy(buf.at[pl.ds(0, 1)],
                             out_hbm.at[b + BATCH, pl.ds(0, 1)], sem),
            pltpu.async_copy(ctxn_v,
                             out_hbm.at[b + BATCH, pl.ds(1, N_CTX)], sem),
            pltpu.async_copy(buf.at[pl.ds(SUF0, N_SUF)],
                             out_hbm.at[b + BATCH, pl.ds(SUF0, N_SUF)], sem),
        )

    def wait(descrs):
        for d in descrs:
            d.wait()

    def body(j, carry):
        la = 4 * j
        a = base + la
        ga = gather(la, buf0, sem_g0)
        gb = gather(la + 1, buf1, sem_g1)
        wait(ga)
        wa = write(a, buf0, sem_w0)
        wait(gb)
        wb = write(a + 1, buf1, sem_w1)
        wait(wa)
        gc = gather(la + 2, buf0, sem_g0)
        wait(wb)
        gd = gather(la + 3, buf1, sem_g1)
        wait(gc)
        wc = write(a + 2, buf0, sem_w0)
        wait(gd)
        wd = write(a + 3, buf1, sem_w1)
        wait(wc)
        wait(wd)
        return carry

    lax.fori_loop(0, B_PER_W // 4, body, 0)


def kernel(tokenized_prompts, token_embedding, ctx_pos, ctx_neg):
    mesh = plsc.VectorSubcoreMesh(core_axis_name="c", subcore_axis_name="s")
    f = functools.partial(
        pl.kernel,
        mesh=mesh,
        compiler_params=pltpu.CompilerParams(use_tc_tiling_on_sc=False),
        out_type=(
            jax.ShapeDtypeStruct((2 * BATCH, CTX_LEN, CTX_DIM), jnp.float32),
            jax.ShapeDtypeStruct((2 * BATCH, CTX_LEN), jnp.int32),
        ),
        scratch_types=[
            pltpu.VMEM((B_PER_W, CTX_LEN), jnp.int32),
            pltpu.VMEM((B_PER_W, 1), jnp.int32),
            pltpu.VMEM((B_PER_W, N_SUF), jnp.int32),
            pltpu.VMEM((CTX_LEN, CTX_DIM), jnp.float32),
            pltpu.VMEM((CTX_LEN, CTX_DIM), jnp.float32),
            pltpu.VMEM((N_CTX, CTX_DIM), jnp.float32),
            pltpu.SemaphoreType.DMA,
            pltpu.SemaphoreType.DMA,
            pltpu.SemaphoreType.DMA,
            pltpu.SemaphoreType.DMA,
        ],
    )(_prompt_body)
    sos_idx = tokenized_prompts[:, :1]
    suf_idx = tokenized_prompts[:, 1 + N_CTX:]
    return f(tokenized_prompts, sos_idx, suf_idx,
             token_embedding, ctx_pos, ctx_neg)
